# baseline (speedup 1.0000x reference)
import functools

import jax
import jax.numpy as jnp
from jax import lax
from jax.experimental import pallas as pl
from jax.experimental.pallas import tpu as pltpu
from jax.experimental.pallas import tpu_sc as plsc

D = 16
B = 16384
NF = 16
TSIZES = [1000001, 100001, 102, 3, 32, 13, 4, 5, 45001, 31, 51, 251, 5, 5,
          113, 300001]
BIG = (0, 1, 8, 15)
SMALL = tuple(f for f in range(NF) if f not in BIG)
_SOFF = {}
_off = 0
for _f in SMALL:
    _SOFF[_f] = _off
    _off += TSIZES[_f]
SMALL_W = 640

_info = plsc.get_sparse_core_info()
NC, NS = _info.num_cores, _info.num_subcores
NW = NC * NS
CHUNK = B // NW
SUB = 128
NSUB = CHUNK // SUB

TCW = 16384


def _tc_repack(in_ref, out_ref):
    x = in_ref[...]
    xt3 = x.T.reshape(TCW // 8, 8, 16)
    out_ref[...] = jnp.concatenate([xt3[:, s, :] for s in range(8)], axis=1)


def _repack(tabT):
    s = tabT.shape[1]
    g = (s + TCW - 1) // TCW
    return pl.pallas_call(
        _tc_repack,
        grid=(g,),
        in_specs=[pl.BlockSpec((16, TCW), lambda i: (0, i))],
        out_specs=pl.BlockSpec((TCW // 8, 128), lambda i: (i, 0)),
        out_shape=jax.ShapeDtypeStruct((g * TCW // 8, 128), jnp.float32),
    )(tabT)


def _sc_kernel(xT, smallcat, *rest):
    bigs = rest[:4]
    res = rest[4]
    idx3, jb3, staged, gbufa, gbufb, buf, sem = rest[5:]

    wid = lax.axis_index("s") * NC + lax.axis_index("c")
    base = wid * CHUNK
    iota = lax.iota(jnp.int32, 16)

    for j in range(NSUB):
        pltpu.sync_copy(xT.at[:, pl.ds(base + j * SUB, SUB)], idx3.at[:, j])
    pltpu.sync_copy(smallcat, staged)

    def jbody(v, carry):
        jj = v >> 3
        g16 = (v & 7) * 16
        for bi, f in enumerate(BIG):
            jb3[bi, jj, pl.ds(g16, 16)] = idx3[f, jj, pl.ds(g16, 16)] >> 3
        return carry

    lax.fori_loop(0, NSUB * 8, jbody, 0)

    def extract(f, j, gbuf):
        def ebody(g, carry):
            b0 = j * SUB + g * 16
            rvec = idx3[f, j, pl.ds(g * 16, 16)]
            colbase = (rvec & 7) * 16
            rowvec = g * 16 + iota
            for d in range(D):
                buf[d, pl.ds(b0, 16)] = plsc.load_gather(
                    gbuf, [rowvec, colbase + d])
            return carry

        lax.fori_loop(0, SUB // 16, ebody, 0)

    for bi, f in enumerate(BIG):
        tab = bigs[bi]
        gbufs = (gbufa, gbufb)
        copies = [None, None]
        copies[0] = pltpu.async_copy(tab.at[jb3.at[bi, 0]], gbufa, sem)
        for j in range(NSUB):
            if j + 1 < NSUB:
                copies[(j + 1) % 2] = pltpu.async_copy(
                    tab.at[jb3.at[bi, j + 1]], gbufs[(j + 1) % 2], sem)
            copies[j % 2].wait()
            extract(f, j, gbufs[j % 2])
        pltpu.sync_copy(buf, res.at[f, :, pl.ds(base, CHUNK)])

    for f in SMALL:
        off = _SOFF[f]

        def sbody(g, carry, f=f, off=off):
            rv = idx3[f, g >> 3, pl.ds((g & 7) * 16, 16)] + off
            for d in range(D):
                dv = jnp.full((16,), d, jnp.int32)
                buf[d, pl.ds(g * 16, 16)] = plsc.load_gather(staged, [dv, rv])
            return carry

        lax.fori_loop(0, CHUNK // 16, sbody, 0)
        pltpu.sync_copy(buf, res.at[f, :, pl.ds(base, CHUNK)])


def _run_sc(xT, smallcat, *bigs):
    mesh = plsc.VectorSubcoreMesh(core_axis_name="c", subcore_axis_name="s")
    k = functools.partial(
        pl.kernel,
        mesh=mesh,
        out_type=jax.ShapeDtypeStruct((NF, D, B), jnp.float32),
        scratch_types=[
            pltpu.VMEM((NF, NSUB, SUB), jnp.int32),
            pltpu.VMEM((len(BIG), NSUB, SUB), jnp.int32),
            pltpu.VMEM((D, SMALL_W), jnp.float32),
            pltpu.VMEM((SUB, SUB), jnp.float32),
            pltpu.VMEM((SUB, SUB), jnp.float32),
            pltpu.VMEM((D, CHUNK), jnp.float32),
            pltpu.SemaphoreType.DMA,
        ],
        compiler_params=pltpu.CompilerParams(needs_layout_passes=False),
    )(_sc_kernel)
    return k(xT, smallcat, *bigs)


def kernel(x, table_0, table_1, table_2, table_3, table_4, table_5, table_6,
           table_7, table_8, table_9, table_10, table_11, table_12, table_13,
           table_14, table_15):
    tabs = [table_0, table_1, table_2, table_3, table_4, table_5, table_6,
            table_7, table_8, table_9, table_10, table_11, table_12, table_13,
            table_14, table_15]
    smallcat = jnp.concatenate([tabs[f].T for f in SMALL], axis=1)
    smallcat = jnp.pad(smallcat, ((0, 0), (0, SMALL_W - smallcat.shape[1])))
    bigs = [_repack(tabs[f].T) for f in BIG]
    res = _run_sc(x.T, smallcat, *bigs)
    return res.transpose(2, 0, 1)


# speedup vs baseline: 1.0170x; 1.0170x over previous
import functools

import jax
import jax.numpy as jnp
from jax import lax
from jax.experimental import pallas as pl
from jax.experimental.pallas import tpu as pltpu
from jax.experimental.pallas import tpu_sc as plsc

D = 16
B = 16384
NF = 16
TSIZES = [1000001, 100001, 102, 3, 32, 13, 4, 5, 45001, 31, 51, 251, 5, 5,
          113, 300001]
BIG = (0, 1, 8, 15)
SMALL = tuple(f for f in range(NF) if f not in BIG)
_SOFF = {}
_off = 0
for _f in SMALL:
    _SOFF[_f] = _off
    _off += TSIZES[_f]
SMALL_W = 640

_info = plsc.get_sparse_core_info()
NC, NS = _info.num_cores, _info.num_subcores
NW = NC * NS
CHUNK = B // NW
SUB = 128
NSUB = CHUNK // SUB

TCW = 8192


def _tc_repack(in_ref, out_ref):
    x = in_ref[...]
    xt3 = x.T.reshape(TCW // 8, 8, 16)
    out_ref[...] = jnp.concatenate([xt3[:, s, :] for s in range(8)], axis=1)


def _repack(tabT):
    s = tabT.shape[1]
    g = (s + TCW - 1) // TCW
    return pl.pallas_call(
        _tc_repack,
        grid=(g,),
        in_specs=[pl.BlockSpec((16, TCW), lambda i: (0, i))],
        out_specs=pl.BlockSpec((TCW // 8, 128), lambda i: (i, 0)),
        out_shape=jax.ShapeDtypeStruct((g * TCW // 8, 128), jnp.float32),
    )(tabT)


def _sc_kernel(xT, smallcat, *rest):
    bigs = rest[:4]
    res = rest[4]
    idx3, jb3, staged, gbufa, gbufb, buf, sem = rest[5:]

    wid = lax.axis_index("s") * NC + lax.axis_index("c")
    base = wid * CHUNK
    iota = lax.iota(jnp.int32, 16)

    for j in range(NSUB):
        pltpu.sync_copy(xT.at[:, pl.ds(base + j * SUB, SUB)], idx3.at[:, j])
    pltpu.sync_copy(smallcat, staged)

    def jbody(v, carry):
        jj = v >> 3
        g16 = (v & 7) * 16
        for bi, f in enumerate(BIG):
            jb3[bi, jj, pl.ds(g16, 16)] = idx3[f, jj, pl.ds(g16, 16)] >> 3
        return carry

    lax.fori_loop(0, NSUB * 8, jbody, 0)

    def extract(f, j, gbuf):
        def ebody(g, carry):
            b0 = j * SUB + g * 16
            rvec = idx3[f, j, pl.ds(g * 16, 16)]
            colbase = (rvec & 7) * 16
            rowvec = g * 16 + iota
            for d in range(D):
                buf[d, pl.ds(b0, 16)] = plsc.load_gather(
                    gbuf, [rowvec, colbase + d])
            return carry

        lax.fori_loop(0, SUB // 16, ebody, 0)

    for bi, f in enumerate(BIG):
        tab = bigs[bi]
        gbufs = (gbufa, gbufb)
        copies = [None, None]
        copies[0] = pltpu.async_copy(tab.at[jb3.at[bi, 0]], gbufa, sem)
        for j in range(NSUB):
            if j + 1 < NSUB:
                copies[(j + 1) % 2] = pltpu.async_copy(
                    tab.at[jb3.at[bi, j + 1]], gbufs[(j + 1) % 2], sem)
            copies[j % 2].wait()
            extract(f, j, gbufs[j % 2])
        pltpu.sync_copy(buf, res.at[f, :, pl.ds(base, CHUNK)])

    for f in SMALL:
        off = _SOFF[f]

        def sbody(g, carry, f=f, off=off):
            rv = idx3[f, g >> 3, pl.ds((g & 7) * 16, 16)] + off
            for d in range(D):
                dv = jnp.full((16,), d, jnp.int32)
                buf[d, pl.ds(g * 16, 16)] = plsc.load_gather(staged, [dv, rv])
            return carry

        lax.fori_loop(0, CHUNK // 16, sbody, 0)
        pltpu.sync_copy(buf, res.at[f, :, pl.ds(base, CHUNK)])


def _run_sc(xT, smallcat, *bigs):
    mesh = plsc.VectorSubcoreMesh(core_axis_name="c", subcore_axis_name="s")
    k = functools.partial(
        pl.kernel,
        mesh=mesh,
        out_type=jax.ShapeDtypeStruct((NF, D, B), jnp.float32),
        scratch_types=[
            pltpu.VMEM((NF, NSUB, SUB), jnp.int32),
            pltpu.VMEM((len(BIG), NSUB, SUB), jnp.int32),
            pltpu.VMEM((D, SMALL_W), jnp.float32),
            pltpu.VMEM((SUB, SUB), jnp.float32),
            pltpu.VMEM((SUB, SUB), jnp.float32),
            pltpu.VMEM((D, CHUNK), jnp.float32),
            pltpu.SemaphoreType.DMA,
        ],
        compiler_params=pltpu.CompilerParams(needs_layout_passes=False),
    )(_sc_kernel)
    return k(xT, smallcat, *bigs)


def kernel(x, table_0, table_1, table_2, table_3, table_4, table_5, table_6,
           table_7, table_8, table_9, table_10, table_11, table_12, table_13,
           table_14, table_15):
    tabs = [table_0, table_1, table_2, table_3, table_4, table_5, table_6,
            table_7, table_8, table_9, table_10, table_11, table_12, table_13,
            table_14, table_15]
    smallcat = jnp.concatenate([tabs[f].T for f in SMALL], axis=1)
    smallcat = jnp.pad(smallcat, ((0, 0), (0, SMALL_W - smallcat.shape[1])))
    bigs = [_repack(tabs[f].T) for f in BIG]
    res = _run_sc(x.T, smallcat, *bigs)
    return res.transpose(2, 0, 1)


# speedup vs baseline: 1.8066x; 1.7763x over previous
import functools

import jax
import jax.numpy as jnp
from jax import lax
from jax.experimental import pallas as pl
from jax.experimental.pallas import tpu as pltpu
from jax.experimental.pallas import tpu_sc as plsc

D = 16
B = 16384
NF = 16
TSIZES = [1000001, 100001, 102, 3, 32, 13, 4, 5, 45001, 31, 51, 251, 5, 5,
          113, 300001]
BIG = (0, 1, 8, 15)
BIG_TC = (1, 8, 15)
SMALL = tuple(f for f in range(NF) if f not in BIG)
_SOFF = {}
_off = 0
for _f in SMALL:
    _SOFF[_f] = _off
    _off += TSIZES[_f]
SMALL_W = 640

_info = plsc.get_sparse_core_info()
NC, NS = _info.num_cores, _info.num_subcores
NW = NC * NS
CHUNK = B // NW
SUB = 128
NSUB = CHUNK // SUB

TCW = 8192


def _tc_repack(in_ref, out_ref):
    x = in_ref[...]
    xt3 = x.T.reshape(TCW // 8, 8, 16)
    out_ref[...] = jnp.concatenate([xt3[:, s, :] for s in range(8)], axis=1)


def _repack(tabT):
    s = tabT.shape[1]
    g = (s + TCW - 1) // TCW
    return pl.pallas_call(
        _tc_repack,
        grid=(g,),
        in_specs=[pl.BlockSpec((16, TCW), lambda i: (0, i))],
        out_specs=pl.BlockSpec((TCW // 8, 128), lambda i: (i, 0)),
        out_shape=jax.ShapeDtypeStruct((g * TCW // 8, 128), jnp.float32),
    )(tabT)


def _sc_kernel(xT, smallcat, tab0T, *rest):
    bigs = rest[:3]
    res = rest[3]
    idx3, jb3, staged, gbufa, gbufb, buf, tbuf, sem = rest[4:]

    wid = lax.axis_index("s") * NC + lax.axis_index("c")
    base = wid * CHUNK
    iota = lax.iota(jnp.int32, 16)

    for j in range(NSUB):
        pltpu.sync_copy(xT.at[:, pl.ds(base + j * SUB, SUB)], idx3.at[:, j])
    pltpu.sync_copy(smallcat, staged)

    def jbody(v, carry):
        jj = v >> 3
        g16 = (v & 7) * 16
        for bi, f in enumerate(BIG_TC):
            jb3[bi, jj, pl.ds(g16, 16)] = idx3[f, jj, pl.ds(g16, 16)] >> 3
        return carry

    lax.fori_loop(0, NSUB * 8, jbody, 0)

    def t0body(g, carry):
        jj = g >> 3
        g16 = (g & 7) * 16
        rvec = idx3[0, jj, pl.ds(g16, 16)]
        cv = (rvec >> 7) * SUB
        cov = rvec & 127
        copies = []
        for k in range(16):
            c = pl.multiple_of(cv[k], SUB)
            copies.append(pltpu.async_copy(
                tab0T.at[:, pl.ds(c, SUB)], tbuf.at[k], sem))
        for cp in copies:
            cp.wait()
        for k in range(16):
            kv = jnp.full((16,), k, jnp.int32)
            cok = jnp.full((16,), cov[k], jnp.int32)
            vals = plsc.load_gather(tbuf, [kv, iota, cok])
            bv = jnp.full((16,), g * 16 + k, jnp.int32)
            plsc.store_scatter(buf, [iota, bv], vals)
        return carry

    lax.fori_loop(0, CHUNK // 16, t0body, 0)
    pltpu.sync_copy(buf, res.at[0, :, pl.ds(base, CHUNK)])

    def extract(f, j, gbuf):
        def ebody(g, carry):
            b0 = j * SUB + g * 16
            rvec = idx3[f, j, pl.ds(g * 16, 16)]
            colbase = (rvec & 7) * 16
            rowvec = g * 16 + iota
            for d in range(D):
                buf[d, pl.ds(b0, 16)] = plsc.load_gather(
                    gbuf, [rowvec, colbase + d])
            return carry

        lax.fori_loop(0, SUB // 16, ebody, 0)

    for bi, f in enumerate(BIG_TC):
        tab = bigs[bi]
        gbufs = (gbufa, gbufb)
        copies = [None, None]
        copies[0] = pltpu.async_copy(tab.at[jb3.at[bi, 0]], gbufa, sem)
        for j in range(NSUB):
            if j + 1 < NSUB:
                copies[(j + 1) % 2] = pltpu.async_copy(
                    tab.at[jb3.at[bi, j + 1]], gbufs[(j + 1) % 2], sem)
            copies[j % 2].wait()
            extract(f, j, gbufs[j % 2])
        pltpu.sync_copy(buf, res.at[f, :, pl.ds(base, CHUNK)])

    for f in SMALL:
        off = _SOFF[f]

        def sbody(g, carry, f=f, off=off):
            rv = idx3[f, g >> 3, pl.ds((g & 7) * 16, 16)] + off
            for d in range(D):
                dv = jnp.full((16,), d, jnp.int32)
                buf[d, pl.ds(g * 16, 16)] = plsc.load_gather(staged, [dv, rv])
            return carry

        lax.fori_loop(0, CHUNK // 16, sbody, 0)
        pltpu.sync_copy(buf, res.at[f, :, pl.ds(base, CHUNK)])


def _run_sc(xT, smallcat, tab0T, *bigs):
    mesh = plsc.VectorSubcoreMesh(core_axis_name="c", subcore_axis_name="s")
    k = functools.partial(
        pl.kernel,
        mesh=mesh,
        out_type=jax.ShapeDtypeStruct((NF, D, B), jnp.float32),
        scratch_types=[
            pltpu.VMEM((NF, NSUB, SUB), jnp.int32),
            pltpu.VMEM((len(BIG_TC), NSUB, SUB), jnp.int32),
            pltpu.VMEM((D, SMALL_W), jnp.float32),
            pltpu.VMEM((SUB, SUB), jnp.float32),
            pltpu.VMEM((SUB, SUB), jnp.float32),
            pltpu.VMEM((D, CHUNK), jnp.float32),
            pltpu.VMEM((16, D, SUB), jnp.float32),
            pltpu.SemaphoreType.DMA,
        ],
        compiler_params=pltpu.CompilerParams(
            needs_layout_passes=False, disable_bounds_checks=True),
    )(_sc_kernel)
    return k(xT, smallcat, tab0T, *bigs)


def kernel(x, table_0, table_1, table_2, table_3, table_4, table_5, table_6,
           table_7, table_8, table_9, table_10, table_11, table_12, table_13,
           table_14, table_15):
    tabs = [table_0, table_1, table_2, table_3, table_4, table_5, table_6,
            table_7, table_8, table_9, table_10, table_11, table_12, table_13,
            table_14, table_15]
    smallcat = jnp.concatenate([tabs[f].T for f in SMALL], axis=1)
    smallcat = jnp.pad(smallcat, ((0, 0), (0, SMALL_W - smallcat.shape[1])))
    bigs = [_repack(tabs[f].T) for f in BIG_TC]
    res = _run_sc(x.T, smallcat, tabs[0].T, *bigs)
    return res.transpose(2, 0, 1)


# speedup vs baseline: 1.9745x; 1.0930x over previous
import functools

import jax
import jax.numpy as jnp
from jax import lax
from jax.experimental import pallas as pl
from jax.experimental.pallas import tpu as pltpu
from jax.experimental.pallas import tpu_sc as plsc

D = 16
B = 16384
NF = 16
TSIZES = [1000001, 100001, 102, 3, 32, 13, 4, 5, 45001, 31, 51, 251, 5, 5,
          113, 300001]
BIG = (0, 1, 8, 15)
BIG_TC = (1, 8)
BIG_NATIVE = (0, 15)
SMALL = tuple(f for f in range(NF) if f not in BIG)
_SOFF = {}
_off = 0
for _f in SMALL:
    _SOFF[_f] = _off
    _off += TSIZES[_f]
SMALL_W = 640

_info = plsc.get_sparse_core_info()
NC, NS = _info.num_cores, _info.num_subcores
NW = NC * NS
CHUNK = B // NW
SUB = 128
NSUB = CHUNK // SUB

TCW = 8192


def _tc_repack(in_ref, out_ref):
    x = in_ref[...]
    xt3 = x.T.reshape(TCW // 8, 8, 16)
    out_ref[...] = jnp.concatenate([xt3[:, s, :] for s in range(8)], axis=1)


def _repack(tabT):
    s = tabT.shape[1]
    g = (s + TCW - 1) // TCW
    return pl.pallas_call(
        _tc_repack,
        grid=(g,),
        in_specs=[pl.BlockSpec((16, TCW), lambda i: (0, i))],
        out_specs=pl.BlockSpec((TCW // 8, 128), lambda i: (i, 0)),
        out_shape=jax.ShapeDtypeStruct((g * TCW // 8, 128), jnp.float32),
    )(tabT)


def _sc_kernel(xT, smallcat, tab0T, tab15T, *rest):
    bigs = rest[:len(BIG_TC)]
    res = rest[len(BIG_TC)]
    idx3, jb3, staged, gbufa, gbufb, buf, tbuf, sem = rest[len(BIG_TC) + 1:]

    wid = lax.axis_index("s") * NC + lax.axis_index("c")
    base = wid * CHUNK
    iota = lax.iota(jnp.int32, 16)

    for j in range(NSUB):
        pltpu.sync_copy(xT.at[:, pl.ds(base + j * SUB, SUB)], idx3.at[:, j])
    pltpu.sync_copy(smallcat, staged)

    def jbody(v, carry):
        jj = v >> 3
        g16 = (v & 7) * 16
        for bi, f in enumerate(BIG_TC):
            jb3[bi, jj, pl.ds(g16, 16)] = idx3[f, jj, pl.ds(g16, 16)] >> 3
        return carry

    lax.fori_loop(0, NSUB * 8, jbody, 0)

    for f, tab in zip(BIG_NATIVE, (tab0T, tab15T)):

        def tbody(g, carry, f=f, tab=tab):
            jj = g >> 3
            g16 = (g & 7) * 16
            rvec = idx3[f, jj, pl.ds(g16, 16)]
            cv = (rvec >> 7) * SUB
            cov = rvec & 127
            copies = []
            for k in range(16):
                c = pl.multiple_of(cv[k], SUB)
                copies.append(pltpu.async_copy(
                    tab.at[:, pl.ds(c, SUB)], tbuf.at[k], sem))
            for cp in copies:
                cp.wait()
            for k in range(16):
                kv = jnp.full((16,), k, jnp.int32)
                cok = jnp.full((16,), cov[k], jnp.int32)
                vals = plsc.load_gather(tbuf, [kv, iota, cok])
                bv = jnp.full((16,), g * 16 + k, jnp.int32)
                plsc.store_scatter(buf, [iota, bv], vals)
            return carry

        lax.fori_loop(0, CHUNK // 16, tbody, 0)
        pltpu.sync_copy(buf, res.at[f, :, pl.ds(base, CHUNK)])

    def extract(f, j, gbuf):
        def ebody(g, carry):
            b0 = j * SUB + g * 16
            rvec = idx3[f, j, pl.ds(g * 16, 16)]
            colbase = (rvec & 7) * 16
            rowvec = g * 16 + iota
            for d in range(D):
                buf[d, pl.ds(b0, 16)] = plsc.load_gather(
                    gbuf, [rowvec, colbase + d])
            return carry

        lax.fori_loop(0, SUB // 16, ebody, 0)

    for bi, f in enumerate(BIG_TC):
        tab = bigs[bi]
        gbufs = (gbufa, gbufb)
        copies = [None, None]
        copies[0] = pltpu.async_copy(tab.at[jb3.at[bi, 0]], gbufa, sem)
        for j in range(NSUB):
            if j + 1 < NSUB:
                copies[(j + 1) % 2] = pltpu.async_copy(
                    tab.at[jb3.at[bi, j + 1]], gbufs[(j + 1) % 2], sem)
            copies[j % 2].wait()
            extract(f, j, gbufs[j % 2])
        pltpu.sync_copy(buf, res.at[f, :, pl.ds(base, CHUNK)])

    for f in SMALL:
        off = _SOFF[f]

        def sbody(g, carry, f=f, off=off):
            rv = idx3[f, g >> 3, pl.ds((g & 7) * 16, 16)] + off
            for d in range(D):
                dv = jnp.full((16,), d, jnp.int32)
                buf[d, pl.ds(g * 16, 16)] = plsc.load_gather(staged, [dv, rv])
            return carry

        lax.fori_loop(0, CHUNK // 16, sbody, 0)
        pltpu.sync_copy(buf, res.at[f, :, pl.ds(base, CHUNK)])


def _run_sc(xT, smallcat, tab0T, tab15T, *bigs):
    mesh = plsc.VectorSubcoreMesh(core_axis_name="c", subcore_axis_name="s")
    k = functools.partial(
        pl.kernel,
        mesh=mesh,
        out_type=jax.ShapeDtypeStruct((NF, D, B), jnp.float32),
        scratch_types=[
            pltpu.VMEM((NF, NSUB, SUB), jnp.int32),
            pltpu.VMEM((len(BIG_TC), NSUB, SUB), jnp.int32),
            pltpu.VMEM((D, SMALL_W), jnp.float32),
            pltpu.VMEM((SUB, SUB), jnp.float32),
            pltpu.VMEM((SUB, SUB), jnp.float32),
            pltpu.VMEM((D, CHUNK), jnp.float32),
            pltpu.VMEM((16, D, SUB), jnp.float32),
            pltpu.SemaphoreType.DMA,
        ],
        compiler_params=pltpu.CompilerParams(
            needs_layout_passes=False, disable_bounds_checks=True),
    )(_sc_kernel)
    return k(xT, smallcat, tab0T, tab15T, *bigs)


def kernel(x, table_0, table_1, table_2, table_3, table_4, table_5, table_6,
           table_7, table_8, table_9, table_10, table_11, table_12, table_13,
           table_14, table_15):
    tabs = [table_0, table_1, table_2, table_3, table_4, table_5, table_6,
            table_7, table_8, table_9, table_10, table_11, table_12, table_13,
            table_14, table_15]
    smallcat = jnp.concatenate([tabs[f].T for f in SMALL], axis=1)
    smallcat = jnp.pad(smallcat, ((0, 0), (0, SMALL_W - smallcat.shape[1])))
    bigs = [_repack(tabs[f].T) for f in BIG_TC]
    res = _run_sc(x.T, smallcat, tabs[0].T, tabs[15].T, *bigs)
    return res.transpose(2, 0, 1)
